# Initial kernel scaffold; baseline (speedup 1.0000x reference)
import jax
import jax.numpy as jnp
from jax.experimental import pallas as pl


def kernel(deltas, pe):
    raise NotImplementedError("write your pallas kernel here")


# speedup vs baseline: 10.4151x; 10.4151x over previous
import functools

import jax
import jax.numpy as jnp
from jax import lax
from jax.experimental import pallas as pl
from jax.experimental.pallas import tpu as pltpu
from jax.experimental.pallas import tpu_sc as plsc

MAX_LEN = 2048
D_HALF = 64
BATCH = 4096
HIST = 200

M = BATCH * HIST * 2
NUM_WORKERS = 32
CHUNK = M // NUM_WORKERS
K = 512
NBLK = CHUNK // K

_mesh = plsc.VectorSubcoreMesh(core_axis_name="c", subcore_axis_name="s")


@functools.partial(
    pl.kernel,
    mesh=_mesh,
    out_type=jax.ShapeDtypeStruct((M, D_HALF), jnp.float32),
    scratch_types=[
        pltpu.VMEM((K,), jnp.int32),
        pltpu.VMEM((K, D_HALF), jnp.float32),
        pltpu.SemaphoreType.DMA,
    ],
    compiler_params=pltpu.CompilerParams(use_tc_tiling_on_sc=False),
)
def _pe_gather(deltas_hbm, table_hbm, out_hbm, idx_v, rows_v, sem):
    wid = lax.axis_index("s") * 2 + lax.axis_index("c")
    base = wid * CHUNK
    parity = lax.iota(jnp.int32, 16) & 1

    def block(g, carry):
        off = base + g * K
        pltpu.sync_copy(deltas_hbm.at[pl.ds(off, K)], idx_v)

        def transform(t, c):
            v = idx_v[pl.ds(t * 16, 16)]
            idx_v[pl.ds(t * 16, 16)] = v * 2 + parity
            return c

        lax.fori_loop(0, K // 16, transform, 0)
        pltpu.async_copy(table_hbm.at[idx_v], rows_v, sem).wait()
        pltpu.sync_copy(rows_v, out_hbm.at[pl.ds(off, K)])
        return carry

    lax.fori_loop(0, NBLK, block, 0)


def kernel(deltas, pe):
    deltas_flat = deltas.reshape(M)
    table = pe.reshape(MAX_LEN * 2, D_HALF)
    out = _pe_gather(deltas_flat, table)
    return out.reshape(BATCH, HIST, 2 * D_HALF)


# speedup vs baseline: 11.0611x; 1.0620x over previous
import functools

import jax
import jax.numpy as jnp
from jax import lax
from jax.experimental import pallas as pl
from jax.experimental.pallas import tpu as pltpu
from jax.experimental.pallas import tpu_sc as plsc

MAX_LEN = 2048
D_HALF = 64
BATCH = 4096
HIST = 200

M = BATCH * HIST * 2
NUM_WORKERS = 32
CHUNK = M // NUM_WORKERS
K = 512
NBLK = CHUNK // K
NBUF = 2

_mesh = plsc.VectorSubcoreMesh(core_axis_name="c", subcore_axis_name="s")


@functools.partial(
    pl.kernel,
    mesh=_mesh,
    out_type=jax.ShapeDtypeStruct((M, D_HALF), jnp.float32),
    scratch_types=[
        pltpu.VMEM((CHUNK,), jnp.int32),
        [pltpu.VMEM((K, D_HALF), jnp.float32) for _ in range(NBUF)],
        [pltpu.SemaphoreType.DMA for _ in range(NBUF)],
        [pltpu.SemaphoreType.DMA for _ in range(NBUF)],
    ],
    compiler_params=pltpu.CompilerParams(use_tc_tiling_on_sc=False),
)
def _pe_gather(deltas_hbm, table_hbm, out_hbm, idx_v, rows, gsems, wsems):
    wid = lax.axis_index("s") * 2 + lax.axis_index("c")
    base = wid * CHUNK
    pltpu.sync_copy(deltas_hbm.at[pl.ds(base, CHUNK)], idx_v)
    parity = lax.iota(jnp.int32, 16) & 1

    @plsc.parallel_loop(0, CHUNK, 16, unroll=8)
    def _transform(i):
        idx_v[pl.ds(i, 16)] = idx_v[pl.ds(i, 16)] * 2 + parity

    def start_gather(g, r):
        pltpu.async_copy(
            table_hbm.at[idx_v.at[pl.ds(g * K, K)]], rows[r], gsems[r])

    for r in range(NBUF):
        start_gather(r, r)

    def body(t, carry):
        for r in range(NBUF):
            g = t * NBUF + r
            pltpu.make_async_copy(
                table_hbm.at[idx_v.at[pl.ds(0, K)]], rows[r], gsems[r]).wait()
            pltpu.async_copy(
                rows[r], out_hbm.at[pl.ds(base + g * K, K)], wsems[r])
        for r in range(NBUF):
            g_next = t * NBUF + r + NBUF
            pltpu.make_async_copy(
                rows[r], out_hbm.at[pl.ds(base, K)], wsems[r]).wait()

            @pl.when(g_next < NBLK)
            def _():
                start_gather(g_next, r)

        return carry

    lax.fori_loop(0, NBLK // NBUF, body, 0)


def kernel(deltas, pe):
    deltas_flat = deltas.reshape(M)
    table = pe.reshape(MAX_LEN * 2, D_HALF)
    out = _pe_gather(deltas_flat, table)
    return out.reshape(BATCH, HIST, 2 * D_HALF)


# speedup vs baseline: 11.0756x; 1.0013x over previous
import functools

import jax
import jax.numpy as jnp
from jax import lax
from jax.experimental import pallas as pl
from jax.experimental.pallas import tpu as pltpu
from jax.experimental.pallas import tpu_sc as plsc

MAX_LEN = 2048
D_HALF = 64
BATCH = 4096
HIST = 200

M = BATCH * HIST * 2
NUM_WORKERS = 32
CHUNK = M // NUM_WORKERS
K = 256
NBLK = CHUNK // K
NBUF = 4

_mesh = plsc.VectorSubcoreMesh(core_axis_name="c", subcore_axis_name="s")


@functools.partial(
    pl.kernel,
    mesh=_mesh,
    out_type=jax.ShapeDtypeStruct((M, D_HALF), jnp.float32),
    scratch_types=[
        pltpu.VMEM((CHUNK,), jnp.int32),
        [pltpu.VMEM((K, D_HALF), jnp.float32) for _ in range(NBUF)],
        [pltpu.SemaphoreType.DMA for _ in range(NBUF)],
        [pltpu.SemaphoreType.DMA for _ in range(NBUF)],
    ],
    compiler_params=pltpu.CompilerParams(use_tc_tiling_on_sc=False),
)
def _pe_gather(deltas_hbm, table_hbm, out_hbm, idx_v, rows, gsems, wsems):
    wid = lax.axis_index("s") * 2 + lax.axis_index("c")
    base = wid * CHUNK
    pltpu.sync_copy(deltas_hbm.at[pl.ds(base, CHUNK)], idx_v)
    parity = lax.iota(jnp.int32, 16) & 1

    @plsc.parallel_loop(0, CHUNK, 16, unroll=8)
    def _transform(i):
        idx_v[pl.ds(i, 16)] = idx_v[pl.ds(i, 16)] * 2 + parity

    def start_gather(g, r):
        pltpu.async_copy(
            table_hbm.at[idx_v.at[pl.ds(g * K, K)]], rows[r], gsems[r])

    for r in range(NBUF):
        start_gather(r, r)

    def body(t, carry):
        for r in range(NBUF):
            g = t * NBUF + r
            pltpu.make_async_copy(
                table_hbm.at[idx_v.at[pl.ds(0, K)]], rows[r], gsems[r]).wait()
            pltpu.async_copy(
                rows[r], out_hbm.at[pl.ds(base + g * K, K)], wsems[r])
        for r in range(NBUF):
            g_next = t * NBUF + r + NBUF
            pltpu.make_async_copy(
                rows[r], out_hbm.at[pl.ds(base, K)], wsems[r]).wait()

            @pl.when(g_next < NBLK)
            def _():
                start_gather(g_next, r)

        return carry

    lax.fori_loop(0, NBLK // NBUF, body, 0)


def kernel(deltas, pe):
    deltas_flat = deltas.reshape(M)
    table = pe.reshape(MAX_LEN * 2, D_HALF)
    out = _pe_gather(deltas_flat, table)
    return out.reshape(BATCH, HIST, 2 * D_HALF)


# speedup vs baseline: 12.0405x; 1.0871x over previous
import functools

import jax
import jax.numpy as jnp
from jax import lax
from jax.experimental import pallas as pl
from jax.experimental.pallas import tpu as pltpu
from jax.experimental.pallas import tpu_sc as plsc

MAX_LEN = 2048
D_HALF = 64
BATCH = 4096
HIST = 200

M = BATCH * HIST * 2
NUM_WORKERS = 32
CHUNK = M // NUM_WORKERS
K = 256
NBLK = CHUNK // K
NBUF = 2

_mesh = plsc.VectorSubcoreMesh(core_axis_name="c", subcore_axis_name="s")


@functools.partial(
    pl.kernel,
    mesh=_mesh,
    out_type=jax.ShapeDtypeStruct((M, D_HALF), jnp.float32),
    scratch_types=[
        pltpu.VMEM((CHUNK,), jnp.int32),
        [pltpu.VMEM((K, D_HALF), jnp.float32) for _ in range(NBUF)],
        pltpu.MemorySpace.VMEM_SHARED((MAX_LEN * 2, D_HALF), jnp.float32),
        [pltpu.SemaphoreType.DMA for _ in range(NBUF)],
        [pltpu.SemaphoreType.DMA for _ in range(NBUF)],
    ],
    compiler_params=pltpu.CompilerParams(use_tc_tiling_on_sc=False),
)
def _pe_gather(deltas_hbm, table_hbm, out_hbm, idx_v, rows, shared_tab,
               gsems, wsems):
    wid = lax.axis_index("s") * 2 + lax.axis_index("c")
    base = wid * CHUNK
    pltpu.sync_copy(deltas_hbm.at[pl.ds(base, CHUNK)], idx_v)
    parity = lax.iota(jnp.int32, 16) & 1

    @pl.when(lax.axis_index("s") == 0)
    def _stage_table():
        pltpu.sync_copy(table_hbm, shared_tab)

    plsc.subcore_barrier()

    @plsc.parallel_loop(0, CHUNK, 16, unroll=8)
    def _transform(i):
        idx_v[pl.ds(i, 16)] = idx_v[pl.ds(i, 16)] * 2 + parity

    def start_gather(g, r):
        pltpu.async_copy(
            shared_tab.at[idx_v.at[pl.ds(g * K, K)]], rows[r], gsems[r])

    for r in range(NBUF):
        start_gather(r, r)

    def body(t, carry):
        for r in range(NBUF):
            g = t * NBUF + r
            pltpu.make_async_copy(
                shared_tab.at[idx_v.at[pl.ds(0, K)]], rows[r], gsems[r]).wait()
            pltpu.async_copy(
                rows[r], out_hbm.at[pl.ds(base + g * K, K)], wsems[r])
        for r in range(NBUF):
            g_next = t * NBUF + r + NBUF
            pltpu.make_async_copy(
                rows[r], out_hbm.at[pl.ds(base, K)], wsems[r]).wait()

            @pl.when(g_next < NBLK)
            def _():
                start_gather(g_next, r)

        return carry

    lax.fori_loop(0, NBLK // NBUF, body, 0)


def kernel(deltas, pe):
    deltas_flat = deltas.reshape(M)
    table = pe.reshape(MAX_LEN * 2, D_HALF)
    out = _pe_gather(deltas_flat, table)
    return out.reshape(BATCH, HIST, 2 * D_HALF)
